# baseline (speedup 1.0000x reference)
import functools

import jax
import jax.numpy as jnp
from jax import lax
from jax.experimental import pallas as pl
from jax.experimental.pallas import tpu as pltpu
from jax.experimental.pallas import tpu_sc as plsc

_NC = 2
_NS = 16
_NW = _NC * _NS

_B = 4096
_S = 200
_D = 32
_BPW = _B // _NW
_HALF = _S // 2

_mesh = plsc.VectorSubcoreMesh(
    core_axis_name="c", subcore_axis_name="s", num_cores=_NC, num_subcores=_NS
)


_NBUF = 8


@functools.partial(
    pl.kernel,
    out_type=jax.ShapeDtypeStruct((_B, _D), jnp.float32),
    mesh=_mesh,
    compiler_params=pltpu.CompilerParams(use_tc_tiling_on_sc=False),
    scratch_types=[
        pltpu.VMEM((_S, _BPW), jnp.int32),
        [pltpu.VMEM((_BPW, _D), jnp.float32) for _ in range(_NBUF)],
        pltpu.VMEM((_BPW, _D), jnp.float32),
        pltpu.SemaphoreType.DMA,
        [pltpu.SemaphoreType.DMA for _ in range(_NBUF)],
    ],
)
def _pool_sum(idx_hbm, table_hbm, out_hbm, idx_v, bufs, acc_v, isem, gsems):
    cid = lax.axis_index("c")
    sid = lax.axis_index("s")
    wid = sid * _NC + cid

    def stage(s, _):
        pltpu.async_copy(idx_hbm.at[s, wid], idx_v.at[s], isem)
        return 0

    lax.fori_loop(0, _S, stage, 0)

    z = jnp.zeros((16,), jnp.float32)

    def zero(j, _):
        acc_v[j, 0:16] = z
        acc_v[j, 16:32] = z
        return 0

    lax.fori_loop(0, _BPW, zero, 0)

    def stage_wait(s, _):
        pltpu.make_async_copy(idx_hbm.at[s, wid], idx_v.at[s], isem).wait()
        return 0

    lax.fori_loop(0, _S, stage_wait, 0)

    def issue(s, b):
        pltpu.async_copy(table_hbm.at[idx_v.at[s]], bufs[b], gsems[b])

    def drain(b):
        pltpu.make_async_copy(table_hbm.at[pl.ds(0, _BPW)], bufs[b], gsems[b]).wait()

    def accum(b):
        buf = bufs[b]

        def red(j, _):
            base = j * 8
            for u in range(8):
                r = base + u
                plsc.addupdate(acc_v.at[r, pl.ds(0, 16)], buf[r, 0:16])
                plsc.addupdate(acc_v.at[r, pl.ds(16, 16)], buf[r, 16:32])
            return 0

        lax.fori_loop(0, _BPW // 8, red, 0)

    for b in range(_NBUF):
        issue(b, b)

    def body(s0, _):
        for b in range(_NBUF):
            s = s0 * _NBUF + b
            drain(b)
            accum(b)
            issue(s + _NBUF, b)
        return 0

    lax.fori_loop(0, (_S - _NBUF) // _NBUF, body, 0)

    for b in range(_NBUF):
        drain(b)
        accum(b)

    pltpu.sync_copy(acc_v, out_hbm.at[pl.ds(wid * _BPW, _BPW)])


_BB = 512


def _mlp_body(x_ref, w1_ref, b1_ref, w2_ref, b2_ref, o_ref):
    x = x_ref[...] * (1.0 / _S)
    h = jnp.dot(x, w1_ref[...], preferred_element_type=jnp.float32) + b1_ref[...]
    h = jnp.maximum(h, 0.0)
    logits = jnp.dot(h, w2_ref[...], preferred_element_type=jnp.float32) + b2_ref[...]
    m = jnp.max(logits, axis=-1, keepdims=True)
    e = jnp.exp(logits - m)
    o_ref[...] = e / jnp.sum(e, axis=-1, keepdims=True)


def _mlp(pooled_sum, W1, b1, W2, b2):
    n_classes = W2.shape[1]
    hidden = W1.shape[1]
    grid = (_B // _BB,)
    return pl.pallas_call(
        _mlp_body,
        grid=grid,
        in_specs=[
            pl.BlockSpec((_BB, _D), lambda i: (i, 0)),
            pl.BlockSpec((_D, hidden), lambda i: (0, 0)),
            pl.BlockSpec((1, hidden), lambda i: (0, 0)),
            pl.BlockSpec((hidden, n_classes), lambda i: (0, 0)),
            pl.BlockSpec((1, n_classes), lambda i: (0, 0)),
        ],
        out_specs=pl.BlockSpec((_BB, n_classes), lambda i: (i, 0)),
        out_shape=jax.ShapeDtypeStruct((_B, n_classes), jnp.float32),
    )(pooled_sum, W1, b1, W2, b2)


def kernel(inputs, table, W1, b1, W2, b2):
    idx = jnp.swapaxes(inputs.astype(jnp.int32), 0, 1).reshape(_S, _NW, _BPW)
    pooled_sum = _pool_sum(idx, table)
    return _mlp(pooled_sum, W1, b1.reshape(1, -1), W2, b2.reshape(1, -1))
